# Initial kernel scaffold; baseline (speedup 1.0000x reference)
import jax
import jax.numpy as jnp
from jax.experimental import pallas as pl


def kernel(x, node_att, edge_index, edge_feat, edge_att, Wn, bn, Wr, br, Wa, ba):
    raise NotImplementedError("write your pallas kernel here")


# speedup vs baseline: 4.0825x; 4.0825x over previous
import functools

import jax
import jax.numpy as jnp
from jax import lax
from jax.experimental import pallas as pl
from jax.experimental.pallas import tpu as pltpu
from jax.experimental.pallas import tpu_sc as plsc

N = 10000
E = 320000
IN = 128
REL = 16
OUT = 128

NC, NS = 2, 16
NW = NC * NS
EPT = E // NW
K = 80
CH = EPT // K

ROWB = 640
ROWB_LAST = N - ROWB * (NS - 1)
STG = 80

_BLK = 2000
_EBLK = 8000


def _pre_body(x_ref, na_ref, wn_ref, bn_ref, wa1_ref, h_ref, g_ref):
    h = jnp.dot(x_ref[...], wn_ref[...], preferred_element_type=jnp.float32)
    h = h + bn_ref[...]
    h_ref[...] = h
    g_ref[...] = jnp.dot(na_ref[...] * h, wa1_ref[...],
                         preferred_element_type=jnp.float32)


def _tc_pre(x, na2, Wn, bn2, Wa1):
    return pl.pallas_call(
        _pre_body,
        grid=(N // _BLK,),
        in_specs=[
            pl.BlockSpec((_BLK, IN), lambda i: (i, 0)),
            pl.BlockSpec((_BLK, 1), lambda i: (i, 0)),
            pl.BlockSpec((IN, IN), lambda i: (0, 0)),
            pl.BlockSpec((1, IN), lambda i: (0, 0)),
            pl.BlockSpec((IN, OUT), lambda i: (0, 0)),
        ],
        out_specs=[
            pl.BlockSpec((_BLK, IN), lambda i: (i, 0)),
            pl.BlockSpec((_BLK, OUT), lambda i: (i, 0)),
        ],
        out_shape=[
            jax.ShapeDtypeStruct((N, IN), jnp.float32),
            jax.ShapeDtypeStruct((N, OUT), jnp.float32),
        ],
    )(x, na2, Wn, bn2, Wa1)


def _q_body(ef_ref, ea_ref, w2_ref, b2_ref, q_ref):
    q = jnp.dot(ef_ref[...], w2_ref[...], preferred_element_type=jnp.float32)
    q_ref[...] = ea_ref[...] * (q + b2_ref[...])


def _tc_q(ef, ea2, W2, b2):
    return pl.pallas_call(
        _q_body,
        grid=(E // _EBLK,),
        in_specs=[
            pl.BlockSpec((_EBLK, REL), lambda i: (i, 0)),
            pl.BlockSpec((_EBLK, 1), lambda i: (i, 0)),
            pl.BlockSpec((REL, OUT), lambda i: (0, 0)),
            pl.BlockSpec((1, OUT), lambda i: (0, 0)),
        ],
        out_specs=pl.BlockSpec((_EBLK, OUT), lambda i: (i, 0)),
        out_shape=jax.ShapeDtypeStruct((E, OUT), jnp.float32),
    )(ef, ea2, W2, b2)


def _sc_body(g_hbm, q_hbm, src_hbm, dst_hbm, z_hbm,
             p_hbm,
             idx_s, idx_d, rows, qrows, acc):
    c = lax.axis_index("c")
    s = lax.axis_index("s")
    base = (c * NS + s) * EPT

    st = rows.at[pl.ds(0, STG)]

    pltpu.sync_copy(z_hbm.at[pl.ds(0, STG)], st)

    @pl.when(s < NS - 1)
    def _():
        for t in range(ROWB // STG):
            r0 = s * ROWB + t * STG
            pltpu.sync_copy(st, acc.at[pl.ds(r0, STG)])

    @pl.when(s == NS - 1)
    def _():
        for t in range(ROWB_LAST // STG):
            r0 = (NS - 1) * ROWB + t * STG
            pltpu.sync_copy(st, acc.at[pl.ds(r0, STG)])

    plsc.subcore_barrier()

    def chunk(i, carry):
        off = base + i * K
        pltpu.sync_copy(src_hbm.at[pl.ds(off, K)], idx_s)
        pltpu.sync_copy(dst_hbm.at[pl.ds(off, K)], idx_d)
        pltpu.sync_copy(g_hbm.at[idx_s], rows)
        pltpu.sync_copy(q_hbm.at[pl.ds(off, K)], qrows)
        pltpu.sync_copy(rows, acc.at[idx_d], add=True)
        pltpu.sync_copy(qrows, acc.at[idx_d], add=True)
        return carry

    lax.fori_loop(0, CH, chunk, 0)

    plsc.subcore_barrier()

    @pl.when(s < NS - 1)
    def _():
        for t in range(ROWB // STG):
            r0 = s * ROWB + t * STG
            pltpu.sync_copy(acc.at[pl.ds(r0, STG)], st)
            pltpu.sync_copy(st, p_hbm.at[pl.ds(c * N + r0, STG)])

    @pl.when(s == NS - 1)
    def _():
        for t in range(ROWB_LAST // STG):
            r0 = (NS - 1) * ROWB + t * STG
            pltpu.sync_copy(acc.at[pl.ds(r0, STG)], st)
            pltpu.sync_copy(st, p_hbm.at[pl.ds(c * N + r0, STG)])


_sc_segsum = functools.partial(
    pl.kernel,
    out_type=jax.ShapeDtypeStruct((NC * N, OUT), jnp.float32),
    mesh=plsc.VectorSubcoreMesh(core_axis_name="c", subcore_axis_name="s",
                                num_cores=NC, num_subcores=NS),
    scratch_types=[
        pltpu.VMEM((K,), jnp.int32),
        pltpu.VMEM((K,), jnp.int32),
        pltpu.VMEM((K, OUT), jnp.float32),
        pltpu.VMEM((K, OUT), jnp.float32),
        pltpu.VMEM_SHARED((N, OUT), jnp.float32),
    ],
)(_sc_body)


def _post_body(p0_ref, p1_ref, h_ref, na_ref, wa3_ref, ba_ref, o_ref):
    pre = p0_ref[...] + p1_ref[...]
    pre = pre + jnp.dot(h_ref[...], wa3_ref[...],
                        preferred_element_type=jnp.float32)
    pre = pre + ba_ref[...]
    o_ref[...] = na_ref[...] * jnp.maximum(pre, 0.0)


def _tc_post(p, h, na2, Wa3, ba2):
    nb = N // _BLK
    return pl.pallas_call(
        _post_body,
        grid=(nb,),
        in_specs=[
            pl.BlockSpec((_BLK, OUT), lambda i: (i, 0)),
            pl.BlockSpec((_BLK, OUT), lambda i, _nb=nb: (i + _nb, 0)),
            pl.BlockSpec((_BLK, IN), lambda i: (i, 0)),
            pl.BlockSpec((_BLK, 1), lambda i: (i, 0)),
            pl.BlockSpec((IN, OUT), lambda i: (0, 0)),
            pl.BlockSpec((1, OUT), lambda i: (0, 0)),
        ],
        out_specs=pl.BlockSpec((_BLK, OUT), lambda i: (i, 0)),
        out_shape=jax.ShapeDtypeStruct((N, OUT), jnp.float32),
    )(p, p, h, na2, Wa3, ba2)


def kernel(x, node_att, edge_index, edge_feat, edge_att, Wn, bn, Wr, br,
           Wa, ba):
    src = edge_index[0].astype(jnp.int32)
    dst = edge_index[1].astype(jnp.int32)
    na2 = node_att.reshape(N, 1)

    Wa1 = Wa[:IN]
    Wa2 = Wa[IN:IN + REL]
    Wa3 = Wa[IN + REL:]
    W2 = Wr @ Wa2
    b2 = (br @ Wa2).reshape(1, OUT)

    h, g = _tc_pre(x, na2, Wn, bn.reshape(1, IN), Wa1)
    q = _tc_q(edge_feat, edge_att.reshape(E, 1), W2, b2)

    z = jnp.zeros((STG, OUT), jnp.float32)
    p = _sc_segsum(g, q, src, dst, z)

    return _tc_post(p, h, na2, Wa3, ba.reshape(1, OUT))


# speedup vs baseline: 4.8314x; 1.1834x over previous
import functools

import jax
import jax.numpy as jnp
from jax import lax
from jax.experimental import pallas as pl
from jax.experimental.pallas import tpu as pltpu
from jax.experimental.pallas import tpu_sc as plsc

N = 10000
E = 320000
IN = 128
REL = 16
OUT = 128

NC, NS = 2, 16
NW = NC * NS
EPT = E // NW
K = 80
CH = EPT // K

ROWB = 640
ROWB_LAST = N - ROWB * (NS - 1)
STG = 80

_BLK = 2000
_EBLK = 8000


def _pre_body(x_ref, na_ref, wn_ref, bn_ref, wa1_ref, h_ref, g_ref):
    h = jnp.dot(x_ref[...], wn_ref[...], preferred_element_type=jnp.float32)
    h = h + bn_ref[...]
    h_ref[...] = h
    g_ref[...] = jnp.dot(na_ref[...] * h, wa1_ref[...],
                         preferred_element_type=jnp.float32)


def _tc_pre(x, na2, Wn, bn2, Wa1):
    return pl.pallas_call(
        _pre_body,
        grid=(N // _BLK,),
        in_specs=[
            pl.BlockSpec((_BLK, IN), lambda i: (i, 0)),
            pl.BlockSpec((_BLK, 1), lambda i: (i, 0)),
            pl.BlockSpec((IN, IN), lambda i: (0, 0)),
            pl.BlockSpec((1, IN), lambda i: (0, 0)),
            pl.BlockSpec((IN, OUT), lambda i: (0, 0)),
        ],
        out_specs=[
            pl.BlockSpec((_BLK, IN), lambda i: (i, 0)),
            pl.BlockSpec((_BLK, OUT), lambda i: (i, 0)),
        ],
        out_shape=[
            jax.ShapeDtypeStruct((N, IN), jnp.float32),
            jax.ShapeDtypeStruct((N, OUT), jnp.float32),
        ],
    )(x, na2, Wn, bn2, Wa1)


def _q_body(ef_ref, ea_ref, w2_ref, b2_ref, q_ref):
    q = jnp.dot(ef_ref[...], w2_ref[...], preferred_element_type=jnp.float32)
    q_ref[...] = ea_ref[...] * (q + b2_ref[...])


def _tc_q(ef, ea2, W2, b2):
    return pl.pallas_call(
        _q_body,
        grid=(E // _EBLK,),
        in_specs=[
            pl.BlockSpec((_EBLK, REL), lambda i: (i, 0)),
            pl.BlockSpec((_EBLK, 1), lambda i: (i, 0)),
            pl.BlockSpec((REL, OUT), lambda i: (0, 0)),
            pl.BlockSpec((1, OUT), lambda i: (0, 0)),
        ],
        out_specs=pl.BlockSpec((_EBLK, OUT), lambda i: (i, 0)),
        out_shape=jax.ShapeDtypeStruct((E, OUT), jnp.float32),
    )(ef, ea2, W2, b2)


def _sc_body(g_hbm, q_hbm, src_hbm, dst_hbm, z_hbm,
             p_hbm,
             idx_s, idx_d, rows, qrows, acc, sem1, sem2):
    c = lax.axis_index("c")
    s = lax.axis_index("s")
    base = (c * NS + s) * EPT

    st = rows.at[pl.ds(0, STG)]

    pltpu.sync_copy(z_hbm.at[pl.ds(0, STG)], st)

    @pl.when(s < NS - 1)
    def _():
        for t in range(ROWB // STG):
            r0 = s * ROWB + t * STG
            pltpu.sync_copy(st, acc.at[pl.ds(r0, STG)])

    @pl.when(s == NS - 1)
    def _():
        for t in range(ROWB_LAST // STG):
            r0 = (NS - 1) * ROWB + t * STG
            pltpu.sync_copy(st, acc.at[pl.ds(r0, STG)])

    plsc.subcore_barrier()

    def chunk(i, carry):
        off = base + i * K
        d1 = pltpu.async_copy(src_hbm.at[pl.ds(off, K)], idx_s, sem1)
        d2 = pltpu.async_copy(dst_hbm.at[pl.ds(off, K)], idx_d, sem2)
        d1.wait()
        d2.wait()
        d3 = pltpu.async_copy(g_hbm.at[idx_s], rows, sem1)
        d4 = pltpu.async_copy(q_hbm.at[pl.ds(off, K)], qrows, sem2)
        d3.wait()
        d4.wait()
        d5 = pltpu.async_copy(rows, acc.at[idx_d], sem1, add=True)
        d6 = pltpu.async_copy(qrows, acc.at[idx_d], sem2, add=True)
        d5.wait()
        d6.wait()
        return carry

    lax.fori_loop(0, CH, chunk, 0)

    plsc.subcore_barrier()

    @pl.when(s < NS - 1)
    def _():
        for t in range(ROWB // STG):
            r0 = s * ROWB + t * STG
            pltpu.sync_copy(acc.at[pl.ds(r0, STG)], st)
            pltpu.sync_copy(st, p_hbm.at[pl.ds(c * N + r0, STG)])

    @pl.when(s == NS - 1)
    def _():
        for t in range(ROWB_LAST // STG):
            r0 = (NS - 1) * ROWB + t * STG
            pltpu.sync_copy(acc.at[pl.ds(r0, STG)], st)
            pltpu.sync_copy(st, p_hbm.at[pl.ds(c * N + r0, STG)])


_sc_segsum = functools.partial(
    pl.kernel,
    out_type=jax.ShapeDtypeStruct((NC * N, OUT), jnp.float32),
    mesh=plsc.VectorSubcoreMesh(core_axis_name="c", subcore_axis_name="s",
                                num_cores=NC, num_subcores=NS),
    scratch_types=[
        pltpu.VMEM((K,), jnp.int32),
        pltpu.VMEM((K,), jnp.int32),
        pltpu.VMEM((K, OUT), jnp.float32),
        pltpu.VMEM((K, OUT), jnp.float32),
        pltpu.VMEM_SHARED((N, OUT), jnp.float32),
        pltpu.SemaphoreType.DMA,
        pltpu.SemaphoreType.DMA,
    ],
)(_sc_body)


def _post_body(p0_ref, p1_ref, h_ref, na_ref, wa3_ref, ba_ref, o_ref):
    pre = p0_ref[...] + p1_ref[...]
    pre = pre + jnp.dot(h_ref[...], wa3_ref[...],
                        preferred_element_type=jnp.float32)
    pre = pre + ba_ref[...]
    o_ref[...] = na_ref[...] * jnp.maximum(pre, 0.0)


def _tc_post(p, h, na2, Wa3, ba2):
    nb = N // _BLK
    return pl.pallas_call(
        _post_body,
        grid=(nb,),
        in_specs=[
            pl.BlockSpec((_BLK, OUT), lambda i: (i, 0)),
            pl.BlockSpec((_BLK, OUT), lambda i, _nb=nb: (i + _nb, 0)),
            pl.BlockSpec((_BLK, IN), lambda i: (i, 0)),
            pl.BlockSpec((_BLK, 1), lambda i: (i, 0)),
            pl.BlockSpec((IN, OUT), lambda i: (0, 0)),
            pl.BlockSpec((1, OUT), lambda i: (0, 0)),
        ],
        out_specs=pl.BlockSpec((_BLK, OUT), lambda i: (i, 0)),
        out_shape=jax.ShapeDtypeStruct((N, OUT), jnp.float32),
    )(p, p, h, na2, Wa3, ba2)


def kernel(x, node_att, edge_index, edge_feat, edge_att, Wn, bn, Wr, br,
           Wa, ba):
    src = edge_index[0].astype(jnp.int32)
    dst = edge_index[1].astype(jnp.int32)
    na2 = node_att.reshape(N, 1)

    Wa1 = Wa[:IN]
    Wa2 = Wa[IN:IN + REL]
    Wa3 = Wa[IN + REL:]
    W2 = Wr @ Wa2
    b2 = (br @ Wa2).reshape(1, OUT)

    h, g = _tc_pre(x, na2, Wn, bn.reshape(1, IN), Wa1)
    q = _tc_q(edge_feat, edge_att.reshape(E, 1), W2, b2)

    z = jnp.zeros((STG, OUT), jnp.float32)
    p = _sc_segsum(g, q, src, dst, z)

    return _tc_post(p, h, na2, Wa3, ba.reshape(1, OUT))
